# baseline (speedup 1.0000x reference)
import functools

import jax
import jax.numpy as jnp
from jax import lax
from jax.experimental import pallas as pl
from jax.experimental.pallas import tpu as pltpu
from jax.experimental.pallas import tpu_sc as plsc

NE = 10000
NU = 10000
E = 320000
D = 128
Q = 32
NC = 2
NS = 16
CH = 80
SCH = 50
NSC = E // (NS * SCH * CH)
UBASE = 30720
NROW = 40960
TCH_W = E // (NS * NC * CH)
ZPT = NROW // NS
WCH = 128
BR = 2000


def _fill2d(ref, nrows, ncols, value):
    npc = ncols // 16
    if npc == 1:
        @pl.loop(0, nrows)
        def _(r):
            ref[r, pl.ds(0, 16)] = jnp.full((16,), value, dtype=jnp.float32)
    else:
        @pl.loop(0, nrows * npc)
        def _(i):
            r = i // npc
            c = (i % npc) * 16
            ref[r, pl.ds(c, 16)] = jnp.full((16,), value, dtype=jnp.float32)


def _slab_mul_add(dst, a, mult, nrows):
    @pl.loop(0, nrows * (CH // 16))
    def _(i):
        r = i // (CH // 16)
        c = (i % (CH // 16)) * 16
        dst[r, pl.ds(c, 16)] = dst[r, pl.ds(c, 16)] * mult + a[r, pl.ds(c, 16)]


def _slab_add(dst, add, nrows):
    @pl.loop(0, nrows * (CH // 16))
    def _(i):
        r = i // (CH // 16)
        c = (i % (CH // 16)) * 16
        dst[r, pl.ds(c, 16)] = dst[r, pl.ds(c, 16)] + add


_SC_PARAMS = dict(
    compiler_params=pltpu.CompilerParams(use_tc_tiling_on_sc=False,
                                         needs_layout_passes=False),
)

NB = 5
NB1 = 10
NB3 = 4
G3 = 125 // NB3


def _ring_gather_scatter(table, acc, gidx, sidx, gbufs, gsem, ssem,
                         sbufs=None, transform=None):
    nb = len(gbufs)
    for b in range(nb):
        pltpu.async_copy(table.at[gidx.at[b]], gbufs[b], gsem.at[b])
    @pl.loop(0, SCH // nb)
    def _(g):
        for b in range(nb):
            j = g * nb + b
            pltpu.make_async_copy(table.at[gidx.at[j]], gbufs[b],
                                  gsem.at[b]).wait()
            if transform is None:
                src = gbufs[b]
            else:
                src = sbufs[b]
                transform(j, gbufs[b], src)
            pltpu.async_copy(src, acc.at[sidx.at[j]], ssem.at[b], add=True)
            pltpu.make_async_copy(src, acc.at[sidx.at[j]], ssem.at[b]).wait()
            @pl.when(j + nb < SCH)
            def _():
                pltpu.async_copy(table.at[gidx.at[j + nb]], gbufs[b],
                                 gsem.at[b])


def _mesh():
    return plsc.VectorSubcoreMesh(core_axis_name="c", subcore_axis_name="s")


def _k0_body(headS, tyS, uidxS,
             cnt_out,
             buf_a, sidx, ones, bounce_c,
             acc_c):
    cid = lax.axis_index("c")
    sid = lax.axis_index("s")
    zrow0 = sid * ZPT

    _fill2d(ones, CH, 16, 1.0)
    _fill2d(bounce_c, WCH, 16, 0.0)
    @pl.loop(0, ZPT // WCH)
    def _(i):
        pltpu.sync_copy(bounce_c, acc_c.at[pl.ds(zrow0 + i * WCH, WCH)])
    plsc.subcore_barrier()

    @pl.loop(0, NSC)
    def _(sc):
        @pl.when(cid == 0)
        def _():
            pltpu.sync_copy(tyS.at[sid, sc], sidx)
            pltpu.sync_copy(headS.at[sid, sc], buf_a)
            _slab_mul_add(sidx, buf_a, NE, SCH)
        @pl.when(cid == 1)
        def _():
            pltpu.sync_copy(uidxS.at[sid, sc], sidx)
            _slab_add(sidx, UBASE, SCH)
        @pl.loop(0, SCH)
        def _(j):
            pltpu.sync_copy(ones, acc_c.at[sidx.at[j]], add=True)

    plsc.subcore_barrier()
    @pl.when(cid == 0)
    def _():
        @pl.loop(0, UBASE // NS // WCH)
        def _(i):
            r = sid * (UBASE // NS) + i * WCH
            pltpu.sync_copy(acc_c.at[pl.ds(r, WCH)], bounce_c)
            pltpu.sync_copy(bounce_c, cnt_out.at[pl.ds(r, WCH)])
    @pl.when(cid == 1)
    def _():
        @pl.loop(0, (NROW - UBASE) // NS // WCH)
        def _(i):
            r = UBASE + sid * ((NROW - UBASE) // NS) + i * WCH
            pltpu.sync_copy(acc_c.at[pl.ds(r, WCH)], bounce_c)
            pltpu.sync_copy(bounce_c, cnt_out.at[pl.ds(r, WCH)])


def _sc_counts(headS, tyS, uidxS):
    f = functools.partial(
        pl.kernel,
        out_type=jax.ShapeDtypeStruct((NROW, 16), jnp.float32),
        mesh=_mesh(),
        scratch_types=[
            pltpu.VMEM((SCH, CH), jnp.int32),
            pltpu.VMEM((SCH, CH), jnp.int32),
            pltpu.VMEM((CH, 16), jnp.float32),
            pltpu.VMEM((WCH, 16), jnp.float32),
            pltpu.VMEM_SHARED((NROW, 16), jnp.float32),
        ],
        **_SC_PARAMS,
    )(_k0_body)
    return f(headS, tyS, uidxS)


def _k1_body(embQ, headS, tailS, tyS, uidxS, iidxS,
             sumq_out,
             buf_a, sidx, gidx, r0, r1, r2, r3, r4, r5, r6, r7, r8, r9,
             bounce, gsem, ssem,
             acc):
    cid = lax.axis_index("c")
    sid = lax.axis_index("s")
    zrow0 = sid * ZPT
    rowbufs = (r0, r1, r2, r3, r4, r5, r6, r7, r8, r9)

    for p in range(2):
        q = cid * 2 + p

        _fill2d(bounce, WCH, Q, 0.0)
        @pl.loop(0, ZPT // WCH)
        def _(i):
            pltpu.sync_copy(bounce, acc.at[pl.ds(zrow0 + i * WCH, WCH)])
        plsc.subcore_barrier()

        @pl.loop(0, NSC)
        def _(sc):
            pltpu.sync_copy(tyS.at[sid, sc], sidx)
            pltpu.sync_copy(headS.at[sid, sc], buf_a)
            _slab_mul_add(sidx, buf_a, NE, SCH)
            pltpu.sync_copy(tailS.at[sid, sc], gidx)
            _slab_add(gidx, q * NE, SCH)
            _ring_gather_scatter(embQ, acc, gidx, sidx, rowbufs, gsem, ssem)
            pltpu.sync_copy(uidxS.at[sid, sc], sidx)
            _slab_add(sidx, UBASE, SCH)
            pltpu.sync_copy(iidxS.at[sid, sc], gidx)
            _slab_add(gidx, q * NE, SCH)
            _ring_gather_scatter(embQ, acc, gidx, sidx, rowbufs, gsem, ssem)

        plsc.subcore_barrier()
        @pl.loop(0, ZPT // WCH)
        def _(i):
            r = zrow0 + i * WCH
            pltpu.sync_copy(acc.at[pl.ds(r, WCH)], bounce)
            pltpu.sync_copy(bounce, sumq_out.at[q, pl.ds(r, WCH)])
        plsc.subcore_barrier()


def _sc_sums(embQ, headS, tailS, tyS, uidxS, iidxS):
    f = functools.partial(
        pl.kernel,
        out_type=jax.ShapeDtypeStruct((4, NROW, Q), jnp.float32),
        mesh=_mesh(),
        scratch_types=[
            pltpu.VMEM((SCH, CH), jnp.int32),
            pltpu.VMEM((SCH, CH), jnp.int32),
            pltpu.VMEM((SCH, CH), jnp.int32),
            pltpu.VMEM((CH, Q), jnp.float32),
            pltpu.VMEM((CH, Q), jnp.float32),
            pltpu.VMEM((CH, Q), jnp.float32),
            pltpu.VMEM((CH, Q), jnp.float32),
            pltpu.VMEM((CH, Q), jnp.float32),
            pltpu.VMEM((CH, Q), jnp.float32),
            pltpu.VMEM((CH, Q), jnp.float32),
            pltpu.VMEM((CH, Q), jnp.float32),
            pltpu.VMEM((CH, Q), jnp.float32),
            pltpu.VMEM((CH, Q), jnp.float32),
            pltpu.VMEM((WCH, Q), jnp.float32),
            pltpu.SemaphoreType.DMA((NB1,)),
            pltpu.SemaphoreType.DMA((NB1,)),
            pltpu.VMEM_SHARED((NROW, Q), jnp.float32),
        ],
        **_SC_PARAMS,
    )(_k1_body)
    return f(embQ, headS, tailS, tyS, uidxS, iidxS)


def _dot_chunk(ra, rb, sims, j):
    @pl.loop(0, CH // 16)
    def _(r16):
        out = jnp.zeros((16,), jnp.float32)
        for l in range(16):
            r = r16 * 16 + l
            acc = jnp.zeros((16,), jnp.float32)
            for m in range(4):
                va = ra[r, pl.ds(m * 32, 32)]
                vb = rb[r, pl.ds(m * 32, 32)]
                a0, a1 = plsc.unpack(va, format=plsc.PackFormat.INTERLEAVED,
                                     preferred_element_type=jnp.float32)
                bb0, bb1 = plsc.unpack(vb,
                                       format=plsc.PackFormat.INTERLEAVED,
                                       preferred_element_type=jnp.float32)
                acc = acc + a0 * bb0
                acc = acc + a1 * bb1
            s = jnp.sum(acc)
            lane = lax.broadcasted_iota(jnp.int32, (16,), 0)
            out = jnp.where(lane == l, s, out)
        sims[pl.ds(j * CH + r16 * 16, 16)] = out


def _dot_pass(ctab, ntab, cidx, gidx, abufs, bbufs, asem, bsem, sims):
    for b in range(NB3):
        pltpu.async_copy(ctab.at[cidx.at[b]], abufs[b], asem.at[b])
        pltpu.async_copy(ntab.at[gidx.at[b]], bbufs[b], bsem.at[b])
    @pl.loop(0, G3)
    def _(g):
        for b in range(NB3):
            j = g * NB3 + b
            pltpu.make_async_copy(ctab.at[cidx.at[j]], abufs[b],
                                  asem.at[b]).wait()
            pltpu.make_async_copy(ntab.at[gidx.at[j]], bbufs[b],
                                  bsem.at[b]).wait()
            _dot_chunk(abufs[b], bbufs[b], sims, j)
            @pl.when(j + NB3 < TCH_W)
            def _():
                pltpu.async_copy(ctab.at[cidx.at[j + NB3]], abufs[b],
                                 asem.at[b])
                pltpu.async_copy(ntab.at[gidx.at[j + NB3]], bbufs[b],
                                 bsem.at[b])
    for jt in range(G3 * NB3, TCH_W):
        bt = jt % NB3
        pltpu.make_async_copy(ctab.at[cidx.at[jt]], abufs[bt],
                              asem.at[bt]).wait()
        pltpu.make_async_copy(ntab.at[gidx.at[jt]], bbufs[bt],
                              bsem.at[bt]).wait()
        _dot_chunk(abufs[bt], bbufs[bt], sims, jt)


def _k3_body(U_tbl, ee, headW, tailW, tyW, uidxW, iidxW,
             simE_out, simU_out,
             buf_a, cidx, gidx, a0, a1, a2, a3, b0, b1, b2, b3,
             asem, bsem, sims):
    cid = lax.axis_index("c")
    sid = lax.axis_index("s")
    wid = sid * NC + cid
    ebase = wid * (TCH_W * CH)
    abufs = (a0, a1, a2, a3)
    bbufs = (b0, b1, b2, b3)

    pltpu.sync_copy(tyW.at[wid], cidx)
    pltpu.sync_copy(headW.at[wid], buf_a)
    _slab_mul_add(cidx, buf_a, NE, TCH_W)
    pltpu.sync_copy(tailW.at[wid], gidx)
    _dot_pass(U_tbl, ee, cidx, gidx, abufs, bbufs, asem, bsem, sims)
    pltpu.sync_copy(sims, simE_out.at[pl.ds(ebase, TCH_W * CH)])

    pltpu.sync_copy(uidxW.at[wid], cidx)
    _slab_add(cidx, 3 * NE, TCH_W)
    pltpu.sync_copy(iidxW.at[wid], gidx)
    _dot_pass(U_tbl, ee, cidx, gidx, abufs, bbufs, asem, bsem, sims)
    pltpu.sync_copy(sims, simU_out.at[pl.ds(ebase, TCH_W * CH)])


def _sc_sims(U_tbl, ee, headW, tailW, tyW, uidxW, iidxW):
    f = functools.partial(
        pl.kernel,
        out_type=(jax.ShapeDtypeStruct((E,), jnp.float32),
                  jax.ShapeDtypeStruct((E,), jnp.float32)),
        mesh=_mesh(),
        scratch_types=[
            pltpu.VMEM((TCH_W, CH), jnp.int32),
            pltpu.VMEM((TCH_W, CH), jnp.int32),
            pltpu.VMEM((TCH_W, CH), jnp.int32),
            pltpu.VMEM((CH, D), jnp.bfloat16),
            pltpu.VMEM((CH, D), jnp.bfloat16),
            pltpu.VMEM((CH, D), jnp.bfloat16),
            pltpu.VMEM((CH, D), jnp.bfloat16),
            pltpu.VMEM((CH, D), jnp.bfloat16),
            pltpu.VMEM((CH, D), jnp.bfloat16),
            pltpu.VMEM((CH, D), jnp.bfloat16),
            pltpu.VMEM((CH, D), jnp.bfloat16),
            pltpu.SemaphoreType.DMA((NB3,)),
            pltpu.SemaphoreType.DMA((NB3,)),
            pltpu.VMEM((TCH_W * CH,), jnp.float32),
        ],
        **_SC_PARAMS,
    )(_k3_body)
    return f(U_tbl, ee, headW, tailW, tyW, uidxW, iidxW)


def _k4_body(embQ, headS, tailS, tyS, uidxS, iidxS, simES, simUS,
             sumq_out,
             buf_a, sidx, gidx, simsl, g0, g1, g2, g3, g4,
             r0, r1, r2, r3, r4, bounce,
             gsem, ssem,
             acc):
    cid = lax.axis_index("c")
    sid = lax.axis_index("s")
    zrow0 = sid * ZPT
    gbufs = (g0, g1, g2, g3, g4)
    sbufs = (r0, r1, r2, r3, r4)

    def xform(j, g, rows):
        @pl.loop(0, CH // 16)
        def _(r16):
            sv = simsl[j, pl.ds(r16 * 16, 16)]
            for l in range(16):
                r = r16 * 16 + l
                s = sv[l]
                v = g[r, pl.ds(0, 32)]
                lo, hi = plsc.unpack(v, format=plsc.PackFormat.INTERLEAVED,
                                     preferred_element_type=jnp.float32)
                rows[r, pl.ds(0, 16)] = lo * s
                rows[r, pl.ds(16, 16)] = hi * s

    for p in range(2):
        q = cid * 2 + p

        _fill2d(bounce, WCH, Q, 0.0)
        @pl.loop(0, ZPT // WCH)
        def _(i):
            pltpu.sync_copy(bounce, acc.at[pl.ds(zrow0 + i * WCH, WCH)])
        plsc.subcore_barrier()

        @pl.loop(0, NSC)
        def _(sc):
            for ent in range(2):
                if ent == 0:
                    pltpu.sync_copy(tyS.at[sid, sc], sidx)
                    pltpu.sync_copy(headS.at[sid, sc], buf_a)
                    _slab_mul_add(sidx, buf_a, NE, SCH)
                    pltpu.sync_copy(tailS.at[sid, sc], gidx)
                    pltpu.sync_copy(simES.at[sid, sc], simsl)
                else:
                    pltpu.sync_copy(uidxS.at[sid, sc], sidx)
                    _slab_add(sidx, UBASE, SCH)
                    pltpu.sync_copy(iidxS.at[sid, sc], gidx)
                    pltpu.sync_copy(simUS.at[sid, sc], simsl)
                _slab_add(gidx, q * NE, SCH)
                _ring_gather_scatter(embQ, acc, gidx, sidx, gbufs,
                                     gsem, ssem, sbufs=sbufs,
                                     transform=xform)

        plsc.subcore_barrier()
        @pl.loop(0, ZPT // WCH)
        def _(i):
            r = zrow0 + i * WCH
            pltpu.sync_copy(acc.at[pl.ds(r, WCH)], bounce)
            pltpu.sync_copy(bounce, sumq_out.at[q, pl.ds(r, WCH)])
        plsc.subcore_barrier()


def _sc_wsums(embQ, headS, tailS, tyS, uidxS, iidxS, simES, simUS):
    f = functools.partial(
        pl.kernel,
        out_type=jax.ShapeDtypeStruct((4, NROW, Q), jnp.float32),
        mesh=_mesh(),
        scratch_types=[
            pltpu.VMEM((SCH, CH), jnp.int32),
            pltpu.VMEM((SCH, CH), jnp.int32),
            pltpu.VMEM((SCH, CH), jnp.int32),
            pltpu.VMEM((SCH, CH), jnp.float32),
            pltpu.VMEM((CH, Q), jnp.bfloat16),
            pltpu.VMEM((CH, Q), jnp.bfloat16),
            pltpu.VMEM((CH, Q), jnp.bfloat16),
            pltpu.VMEM((CH, Q), jnp.bfloat16),
            pltpu.VMEM((CH, Q), jnp.bfloat16),
            pltpu.VMEM((CH, Q), jnp.float32),
            pltpu.VMEM((CH, Q), jnp.float32),
            pltpu.VMEM((CH, Q), jnp.float32),
            pltpu.VMEM((CH, Q), jnp.float32),
            pltpu.VMEM((CH, Q), jnp.float32),
            pltpu.VMEM((WCH, Q), jnp.float32),
            pltpu.SemaphoreType.DMA((NB,)),
            pltpu.SemaphoreType.DMA((NB,)),
            pltpu.VMEM_SHARED((NROW, Q), jnp.float32),
        ],
        **_SC_PARAMS,
    )(_k4_body)
    return f(embQ, headS, tailS, tyS, uidxS, iidxS, simES, simUS)


def _k2_body(s_ref, c_ref, emb_ref, out_ref):
    s = s_ref[...]
    c = jnp.maximum(c_ref[...][:, 0:1], 1.0)
    mean = s / c
    n2 = jnp.sum(mean * mean, axis=1, keepdims=True)
    u = ((n2 / (n2 + 1.0)) * mean
         / jnp.maximum(jnp.sqrt(n2), 1e-12) + emb_ref[0])
    out_ref[...] = u.astype(jnp.bfloat16)


def _tc_squash(S, C, EMB2):
    nseg = (3 * NE + NU) // BR
    npere = 3 * NE // BR
    return pl.pallas_call(
        _k2_body,
        grid=(nseg,),
        in_specs=[
            pl.BlockSpec((BR, D), lambda b: (b, 0)),
            pl.BlockSpec((BR, 16), lambda b: (b, 0)),
            pl.BlockSpec((1, BR, D), lambda b: (b // npere, b % (NE // BR), 0)),
        ],
        out_specs=pl.BlockSpec((BR, D), lambda b: (b, 0)),
        out_shape=jax.ShapeDtypeStruct((3 * NE + NU, D), jnp.bfloat16),
    )(S, C, EMB2)


def _k5e_body(s_ref, c_ref, emb_ref, w_ref, out_ref):
    c = jnp.maximum(c_ref[...][:, :, 0:1], 1.0)
    m = s_ref[...] / c
    ew = jnp.exp(w_ref[...])
    wn = ew / jnp.sum(ew)
    out_ref[...] = (emb_ref[...] + wn[0, 0] * m[0]
                    + wn[0, 1] * m[1] + wn[0, 2] * m[2])


def _tc_entity(S2e, Ce, ee, w2):
    return pl.pallas_call(
        _k5e_body,
        grid=(NE // BR,),
        in_specs=[
            pl.BlockSpec((3, BR, D), lambda b: (0, b, 0)),
            pl.BlockSpec((3, BR, 16), lambda b: (0, b, 0)),
            pl.BlockSpec((BR, D), lambda b: (b, 0)),
            pl.BlockSpec((1, 3), lambda b: (0, 0)),
        ],
        out_specs=pl.BlockSpec((BR, D), lambda b: (b, 0)),
        out_shape=jax.ShapeDtypeStruct((NE, D), jnp.float32),
    )(S2e, Ce, ee, w2)


def _k5u_body(s_ref, c_ref, emb_ref, out_ref):
    c = jnp.maximum(c_ref[...][:, 0:1], 1.0)
    out_ref[...] = s_ref[...] / c + emb_ref[...]


def _tc_user(S2u, Cu, ue):
    return pl.pallas_call(
        _k5u_body,
        grid=(NU // BR,),
        in_specs=[
            pl.BlockSpec((BR, D), lambda b: (b, 0)),
            pl.BlockSpec((BR, 16), lambda b: (b, 0)),
            pl.BlockSpec((BR, D), lambda b: (b, 0)),
        ],
        out_specs=pl.BlockSpec((BR, D), lambda b: (b, 0)),
        out_shape=jax.ShapeDtypeStruct((NU, D), jnp.float32),
    )(S2u, Cu, ue)


def kernel(entity_emb, user_emb, w, edge_index, edge_type, user_index,
           item_index):
    ee, ue = entity_emb, user_emb
    headS = edge_index[0].reshape(NS, NSC, SCH, CH)
    tailS = edge_index[1].reshape(NS, NSC, SCH, CH)
    tyS = edge_type.reshape(NS, NSC, SCH, CH)
    uidxS = user_index.reshape(NS, NSC, SCH, CH)
    iidxS = item_index.reshape(NS, NSC, SCH, CH)
    headW = edge_index[0].reshape(NS * NC, TCH_W, CH)
    tailW = edge_index[1].reshape(NS * NC, TCH_W, CH)
    tyW = edge_type.reshape(NS * NC, TCH_W, CH)
    uidxW = user_index.reshape(NS * NC, TCH_W, CH)
    iidxW = item_index.reshape(NS * NC, TCH_W, CH)
    embQ = ee.reshape(NE, 4, Q).transpose(1, 0, 2).reshape(4 * NE, Q)

    cnt = _sc_counts(headS, tyS, uidxS)
    sum_q = _sc_sums(embQ, headS, tailS, tyS, uidxS, iidxS)
    Sp = sum_q.transpose(1, 0, 2).reshape(NROW, D)
    S = jnp.concatenate([Sp[:3 * NE], Sp[UBASE:UBASE + NU]])
    C = jnp.concatenate([cnt[:3 * NE], cnt[UBASE:UBASE + NU]])
    EMB2 = jnp.stack([ee, ue])
    U_tbl = _tc_squash(S, C, EMB2)

    ee16 = ee.astype(jnp.bfloat16)
    simE, simU = _sc_sims(U_tbl, ee16, headW, tailW, tyW, uidxW, iidxW)

    ilv = jnp.stack([jnp.arange(16), jnp.arange(16) + 16], axis=1).reshape(32)
    embQ16 = embQ.astype(jnp.bfloat16)[:, ilv]
    sum2_q = _sc_wsums(embQ16, headS, tailS, tyS, uidxS, iidxS,
                       simE.reshape(NS, NSC, SCH, CH),
                       simU.reshape(NS, NSC, SCH, CH))
    S2 = sum2_q.transpose(1, 0, 2).reshape(NROW, D)

    entity_agg = _tc_entity(S2[:3 * NE].reshape(3, NE, D),
                            C[:3 * NE].reshape(3, NE, 16),
                            ee, w.reshape(1, 3))
    user_agg = _tc_user(S2[UBASE:UBASE + NU], C[3 * NE:], ue)
    return (entity_agg, user_agg)


# speedup vs baseline: 1.2027x; 1.2027x over previous
import functools

import jax
import jax.numpy as jnp
from jax import lax
from jax.experimental import pallas as pl
from jax.experimental.pallas import tpu as pltpu
from jax.experimental.pallas import tpu_sc as plsc

NE = 10000
NU = 10000
E = 320000
D = 128
Q = 32
NC = 2
NS = 16
CH = 80
SCH = 50
NSC = E // (NS * SCH * CH)
UBASE = 30720
NROW = 40960
TCH_W = E // (NS * NC * CH)
ZPT = NROW // NS
WCH = 128
BR = 2000


def _fill2d(ref, nrows, ncols, value):
    npc = ncols // 16
    if npc == 1:
        @pl.loop(0, nrows)
        def _(r):
            ref[r, pl.ds(0, 16)] = jnp.full((16,), value, dtype=jnp.float32)
    else:
        @pl.loop(0, nrows * npc)
        def _(i):
            r = i // npc
            c = (i % npc) * 16
            ref[r, pl.ds(c, 16)] = jnp.full((16,), value, dtype=jnp.float32)


def _slab_mul_add(dst, a, mult, nrows):
    @pl.loop(0, nrows * (CH // 16))
    def _(i):
        r = i // (CH // 16)
        c = (i % (CH // 16)) * 16
        dst[r, pl.ds(c, 16)] = dst[r, pl.ds(c, 16)] * mult + a[r, pl.ds(c, 16)]


def _slab_add(dst, add, nrows):
    @pl.loop(0, nrows * (CH // 16))
    def _(i):
        r = i // (CH // 16)
        c = (i % (CH // 16)) * 16
        dst[r, pl.ds(c, 16)] = dst[r, pl.ds(c, 16)] + add


_SC_PARAMS = dict(
    compiler_params=pltpu.CompilerParams(use_tc_tiling_on_sc=False,
                                         needs_layout_passes=False),
)

NB = 5
NB1 = 10
NB3 = 4
G3 = 125 // NB3


def _ring_gather_scatter(table, acc, gidx, sidx, rowbufs, gsem, ssem,
                         scale=None):
    nb = len(rowbufs)
    for b in range(nb):
        pltpu.async_copy(table.at[gidx.at[b]], rowbufs[b], gsem.at[b])
    @pl.loop(0, SCH // nb)
    def _(g):
        for b in range(nb):
            j = g * nb + b
            rows = rowbufs[b]
            pltpu.make_async_copy(table.at[gidx.at[j]], rows,
                                  gsem.at[b]).wait()
            if scale is not None:
                scale(j, rows)
            pltpu.async_copy(rows, acc.at[sidx.at[j]], ssem.at[b], add=True)
            pltpu.make_async_copy(rows, acc.at[sidx.at[j]], ssem.at[b]).wait()
            @pl.when(j + nb < SCH)
            def _():
                pltpu.async_copy(table.at[gidx.at[j + nb]], rows, gsem.at[b])


def _mesh():
    return plsc.VectorSubcoreMesh(core_axis_name="c", subcore_axis_name="s")


def _k0_body(headS, tyS, uidxS,
             cnt_out,
             buf_a, sidx, ones, bounce_c,
             acc_c):
    cid = lax.axis_index("c")
    sid = lax.axis_index("s")
    zrow0 = sid * ZPT

    _fill2d(ones, CH, 16, 1.0)
    _fill2d(bounce_c, WCH, 16, 0.0)
    @pl.loop(0, ZPT // WCH)
    def _(i):
        pltpu.sync_copy(bounce_c, acc_c.at[pl.ds(zrow0 + i * WCH, WCH)])
    plsc.subcore_barrier()

    @pl.loop(0, NSC)
    def _(sc):
        @pl.when(cid == 0)
        def _():
            pltpu.sync_copy(tyS.at[sid, sc], sidx)
            pltpu.sync_copy(headS.at[sid, sc], buf_a)
            _slab_mul_add(sidx, buf_a, NE, SCH)
        @pl.when(cid == 1)
        def _():
            pltpu.sync_copy(uidxS.at[sid, sc], sidx)
            _slab_add(sidx, UBASE, SCH)
        @pl.loop(0, SCH)
        def _(j):
            pltpu.sync_copy(ones, acc_c.at[sidx.at[j]], add=True)

    plsc.subcore_barrier()
    @pl.when(cid == 0)
    def _():
        @pl.loop(0, UBASE // NS // WCH)
        def _(i):
            r = sid * (UBASE // NS) + i * WCH
            pltpu.sync_copy(acc_c.at[pl.ds(r, WCH)], bounce_c)
            pltpu.sync_copy(bounce_c, cnt_out.at[pl.ds(r, WCH)])
    @pl.when(cid == 1)
    def _():
        @pl.loop(0, (NROW - UBASE) // NS // WCH)
        def _(i):
            r = UBASE + sid * ((NROW - UBASE) // NS) + i * WCH
            pltpu.sync_copy(acc_c.at[pl.ds(r, WCH)], bounce_c)
            pltpu.sync_copy(bounce_c, cnt_out.at[pl.ds(r, WCH)])


def _sc_counts(headS, tyS, uidxS):
    f = functools.partial(
        pl.kernel,
        out_type=jax.ShapeDtypeStruct((NROW, 16), jnp.float32),
        mesh=_mesh(),
        scratch_types=[
            pltpu.VMEM((SCH, CH), jnp.int32),
            pltpu.VMEM((SCH, CH), jnp.int32),
            pltpu.VMEM((CH, 16), jnp.float32),
            pltpu.VMEM((WCH, 16), jnp.float32),
            pltpu.VMEM_SHARED((NROW, 16), jnp.float32),
        ],
        **_SC_PARAMS,
    )(_k0_body)
    return f(headS, tyS, uidxS)


def _k1_body(embQ, headS, tailS, tyS, uidxS, iidxS,
             sumq_out,
             buf_a, sidx, gidx, r0, r1, r2, r3, r4, r5, r6, r7, r8, r9,
             bounce, gsem, ssem,
             acc):
    cid = lax.axis_index("c")
    sid = lax.axis_index("s")
    zrow0 = sid * ZPT
    rowbufs = (r0, r1, r2, r3, r4, r5, r6, r7, r8, r9)

    for p in range(2):
        q = cid * 2 + p

        _fill2d(bounce, WCH, Q, 0.0)
        @pl.loop(0, ZPT // WCH)
        def _(i):
            pltpu.sync_copy(bounce, acc.at[pl.ds(zrow0 + i * WCH, WCH)])
        plsc.subcore_barrier()

        @pl.loop(0, NSC)
        def _(sc):
            pltpu.sync_copy(tyS.at[sid, sc], sidx)
            pltpu.sync_copy(headS.at[sid, sc], buf_a)
            _slab_mul_add(sidx, buf_a, NE, SCH)
            pltpu.sync_copy(tailS.at[sid, sc], gidx)
            _slab_add(gidx, q * NE, SCH)
            _ring_gather_scatter(embQ, acc, gidx, sidx, rowbufs, gsem, ssem)
            pltpu.sync_copy(uidxS.at[sid, sc], sidx)
            _slab_add(sidx, UBASE, SCH)
            pltpu.sync_copy(iidxS.at[sid, sc], gidx)
            _slab_add(gidx, q * NE, SCH)
            _ring_gather_scatter(embQ, acc, gidx, sidx, rowbufs, gsem, ssem)

        plsc.subcore_barrier()
        @pl.loop(0, ZPT // WCH)
        def _(i):
            r = zrow0 + i * WCH
            pltpu.sync_copy(acc.at[pl.ds(r, WCH)], bounce)
            pltpu.sync_copy(bounce, sumq_out.at[q, pl.ds(r, WCH)])
        plsc.subcore_barrier()


def _sc_sums(embQ, headS, tailS, tyS, uidxS, iidxS):
    f = functools.partial(
        pl.kernel,
        out_type=jax.ShapeDtypeStruct((4, NROW, Q), jnp.float32),
        mesh=_mesh(),
        scratch_types=[
            pltpu.VMEM((SCH, CH), jnp.int32),
            pltpu.VMEM((SCH, CH), jnp.int32),
            pltpu.VMEM((SCH, CH), jnp.int32),
            pltpu.VMEM((CH, Q), jnp.float32),
            pltpu.VMEM((CH, Q), jnp.float32),
            pltpu.VMEM((CH, Q), jnp.float32),
            pltpu.VMEM((CH, Q), jnp.float32),
            pltpu.VMEM((CH, Q), jnp.float32),
            pltpu.VMEM((CH, Q), jnp.float32),
            pltpu.VMEM((CH, Q), jnp.float32),
            pltpu.VMEM((CH, Q), jnp.float32),
            pltpu.VMEM((CH, Q), jnp.float32),
            pltpu.VMEM((CH, Q), jnp.float32),
            pltpu.VMEM((WCH, Q), jnp.float32),
            pltpu.SemaphoreType.DMA((NB1,)),
            pltpu.SemaphoreType.DMA((NB1,)),
            pltpu.VMEM_SHARED((NROW, Q), jnp.float32),
        ],
        **_SC_PARAMS,
    )(_k1_body)
    return f(embQ, headS, tailS, tyS, uidxS, iidxS)


def _dot_chunk(ra, rb, sims, j):
    @pl.loop(0, CH // 16)
    def _(r16):
        out = jnp.zeros((16,), jnp.float32)
        for l in range(16):
            r = r16 * 16 + l
            acc = jnp.zeros((16,), jnp.float32)
            for m in range(4):
                va = ra[r, pl.ds(m * 32, 32)]
                vb = rb[r, pl.ds(m * 32, 32)]
                a0, a1 = plsc.unpack(va, format=plsc.PackFormat.INTERLEAVED,
                                     preferred_element_type=jnp.float32)
                bb0, bb1 = plsc.unpack(vb,
                                       format=plsc.PackFormat.INTERLEAVED,
                                       preferred_element_type=jnp.float32)
                acc = acc + a0 * bb0
                acc = acc + a1 * bb1
            s = jnp.sum(acc)
            lane = lax.broadcasted_iota(jnp.int32, (16,), 0)
            out = jnp.where(lane == l, s, out)
        sims[pl.ds(j * CH + r16 * 16, 16)] = out


def _dot_pass(ctab, ntab, cidx, gidx, abufs, bbufs, asem, bsem, sims):
    for b in range(NB3):
        pltpu.async_copy(ctab.at[cidx.at[b]], abufs[b], asem.at[b])
        pltpu.async_copy(ntab.at[gidx.at[b]], bbufs[b], bsem.at[b])
    @pl.loop(0, G3)
    def _(g):
        for b in range(NB3):
            j = g * NB3 + b
            pltpu.make_async_copy(ctab.at[cidx.at[j]], abufs[b],
                                  asem.at[b]).wait()
            pltpu.make_async_copy(ntab.at[gidx.at[j]], bbufs[b],
                                  bsem.at[b]).wait()
            _dot_chunk(abufs[b], bbufs[b], sims, j)
            @pl.when(j + NB3 < TCH_W)
            def _():
                pltpu.async_copy(ctab.at[cidx.at[j + NB3]], abufs[b],
                                 asem.at[b])
                pltpu.async_copy(ntab.at[gidx.at[j + NB3]], bbufs[b],
                                 bsem.at[b])
    for jt in range(G3 * NB3, TCH_W):
        bt = jt % NB3
        pltpu.make_async_copy(ctab.at[cidx.at[jt]], abufs[bt],
                              asem.at[bt]).wait()
        pltpu.make_async_copy(ntab.at[gidx.at[jt]], bbufs[bt],
                              bsem.at[bt]).wait()
        _dot_chunk(abufs[bt], bbufs[bt], sims, jt)


def _k3_body(U_tbl, ee, headW, tailW, tyW, uidxW, iidxW,
             simE_out, simU_out,
             buf_a, cidx, gidx, a0, a1, a2, a3, b0, b1, b2, b3,
             asem, bsem, sims):
    cid = lax.axis_index("c")
    sid = lax.axis_index("s")
    wid = sid * NC + cid
    ebase = wid * (TCH_W * CH)
    abufs = (a0, a1, a2, a3)
    bbufs = (b0, b1, b2, b3)

    pltpu.sync_copy(tyW.at[wid], cidx)
    pltpu.sync_copy(headW.at[wid], buf_a)
    _slab_mul_add(cidx, buf_a, NE, TCH_W)
    pltpu.sync_copy(tailW.at[wid], gidx)
    _dot_pass(U_tbl, ee, cidx, gidx, abufs, bbufs, asem, bsem, sims)
    pltpu.sync_copy(sims, simE_out.at[pl.ds(ebase, TCH_W * CH)])

    pltpu.sync_copy(uidxW.at[wid], cidx)
    _slab_add(cidx, 3 * NE, TCH_W)
    pltpu.sync_copy(iidxW.at[wid], gidx)
    _dot_pass(U_tbl, ee, cidx, gidx, abufs, bbufs, asem, bsem, sims)
    pltpu.sync_copy(sims, simU_out.at[pl.ds(ebase, TCH_W * CH)])


def _sc_sims(U_tbl, ee, headW, tailW, tyW, uidxW, iidxW):
    f = functools.partial(
        pl.kernel,
        out_type=(jax.ShapeDtypeStruct((E,), jnp.float32),
                  jax.ShapeDtypeStruct((E,), jnp.float32)),
        mesh=_mesh(),
        scratch_types=[
            pltpu.VMEM((TCH_W, CH), jnp.int32),
            pltpu.VMEM((TCH_W, CH), jnp.int32),
            pltpu.VMEM((TCH_W, CH), jnp.int32),
            pltpu.VMEM((CH, D), jnp.bfloat16),
            pltpu.VMEM((CH, D), jnp.bfloat16),
            pltpu.VMEM((CH, D), jnp.bfloat16),
            pltpu.VMEM((CH, D), jnp.bfloat16),
            pltpu.VMEM((CH, D), jnp.bfloat16),
            pltpu.VMEM((CH, D), jnp.bfloat16),
            pltpu.VMEM((CH, D), jnp.bfloat16),
            pltpu.VMEM((CH, D), jnp.bfloat16),
            pltpu.SemaphoreType.DMA((NB3,)),
            pltpu.SemaphoreType.DMA((NB3,)),
            pltpu.VMEM((TCH_W * CH,), jnp.float32),
        ],
        **_SC_PARAMS,
    )(_k3_body)
    return f(U_tbl, ee, headW, tailW, tyW, uidxW, iidxW)


def _k4_body(embQ, headS, tailS, tyS, uidxS, iidxS, simES, simUS,
             sumq_out,
             buf_a, sidx, gidx, simsl, r0, r1, r2, r3, r4, bounce,
             gsem, ssem,
             acc):
    cid = lax.axis_index("c")
    sid = lax.axis_index("s")
    zrow0 = sid * ZPT
    rowbufs = (r0, r1, r2, r3, r4)

    def scale(j, rows):
        @pl.loop(0, CH // 16)
        def _(r16):
            sv = simsl[j, pl.ds(r16 * 16, 16)]
            for l in range(16):
                r = r16 * 16 + l
                s = sv[l]
                rows[r, pl.ds(0, 16)] = rows[r, pl.ds(0, 16)] * s
                rows[r, pl.ds(16, 16)] = rows[r, pl.ds(16, 16)] * s

    for p in range(2):
        q = cid * 2 + p

        _fill2d(bounce, WCH, Q, 0.0)
        @pl.loop(0, ZPT // WCH)
        def _(i):
            pltpu.sync_copy(bounce, acc.at[pl.ds(zrow0 + i * WCH, WCH)])
        plsc.subcore_barrier()

        @pl.loop(0, NSC)
        def _(sc):
            for ent in range(2):
                if ent == 0:
                    pltpu.sync_copy(tyS.at[sid, sc], sidx)
                    pltpu.sync_copy(headS.at[sid, sc], buf_a)
                    _slab_mul_add(sidx, buf_a, NE, SCH)
                    pltpu.sync_copy(tailS.at[sid, sc], gidx)
                    pltpu.sync_copy(simES.at[sid, sc], simsl)
                else:
                    pltpu.sync_copy(uidxS.at[sid, sc], sidx)
                    _slab_add(sidx, UBASE, SCH)
                    pltpu.sync_copy(iidxS.at[sid, sc], gidx)
                    pltpu.sync_copy(simUS.at[sid, sc], simsl)
                _slab_add(gidx, q * NE, SCH)
                _ring_gather_scatter(embQ, acc, gidx, sidx, rowbufs,
                                     gsem, ssem, scale=scale)

        plsc.subcore_barrier()
        @pl.loop(0, ZPT // WCH)
        def _(i):
            r = zrow0 + i * WCH
            pltpu.sync_copy(acc.at[pl.ds(r, WCH)], bounce)
            pltpu.sync_copy(bounce, sumq_out.at[q, pl.ds(r, WCH)])
        plsc.subcore_barrier()


def _sc_wsums(embQ, headS, tailS, tyS, uidxS, iidxS, simES, simUS):
    f = functools.partial(
        pl.kernel,
        out_type=jax.ShapeDtypeStruct((4, NROW, Q), jnp.float32),
        mesh=_mesh(),
        scratch_types=[
            pltpu.VMEM((SCH, CH), jnp.int32),
            pltpu.VMEM((SCH, CH), jnp.int32),
            pltpu.VMEM((SCH, CH), jnp.int32),
            pltpu.VMEM((SCH, CH), jnp.float32),
            pltpu.VMEM((CH, Q), jnp.float32),
            pltpu.VMEM((CH, Q), jnp.float32),
            pltpu.VMEM((CH, Q), jnp.float32),
            pltpu.VMEM((CH, Q), jnp.float32),
            pltpu.VMEM((CH, Q), jnp.float32),
            pltpu.VMEM((WCH, Q), jnp.float32),
            pltpu.SemaphoreType.DMA((NB,)),
            pltpu.SemaphoreType.DMA((NB,)),
            pltpu.VMEM_SHARED((NROW, Q), jnp.float32),
        ],
        **_SC_PARAMS,
    )(_k4_body)
    return f(embQ, headS, tailS, tyS, uidxS, iidxS, simES, simUS)


def _k2_body(s_ref, c_ref, emb_ref, out_ref):
    s = s_ref[...]
    c = jnp.maximum(c_ref[...][:, 0:1], 1.0)
    mean = s / c
    n2 = jnp.sum(mean * mean, axis=1, keepdims=True)
    u = ((n2 / (n2 + 1.0)) * mean
         / jnp.maximum(jnp.sqrt(n2), 1e-12) + emb_ref[0])
    out_ref[...] = u.astype(jnp.bfloat16)


def _tc_squash(S, C, EMB2):
    nseg = (3 * NE + NU) // BR
    npere = 3 * NE // BR
    return pl.pallas_call(
        _k2_body,
        grid=(nseg,),
        in_specs=[
            pl.BlockSpec((BR, D), lambda b: (b, 0)),
            pl.BlockSpec((BR, 16), lambda b: (b, 0)),
            pl.BlockSpec((1, BR, D), lambda b: (b // npere, b % (NE // BR), 0)),
        ],
        out_specs=pl.BlockSpec((BR, D), lambda b: (b, 0)),
        out_shape=jax.ShapeDtypeStruct((3 * NE + NU, D), jnp.bfloat16),
    )(S, C, EMB2)


def _k5e_body(s_ref, c_ref, emb_ref, w_ref, out_ref):
    c = jnp.maximum(c_ref[...][:, :, 0:1], 1.0)
    m = s_ref[...] / c
    ew = jnp.exp(w_ref[...])
    wn = ew / jnp.sum(ew)
    out_ref[...] = (emb_ref[...] + wn[0, 0] * m[0]
                    + wn[0, 1] * m[1] + wn[0, 2] * m[2])


def _tc_entity(S2e, Ce, ee, w2):
    return pl.pallas_call(
        _k5e_body,
        grid=(NE // BR,),
        in_specs=[
            pl.BlockSpec((3, BR, D), lambda b: (0, b, 0)),
            pl.BlockSpec((3, BR, 16), lambda b: (0, b, 0)),
            pl.BlockSpec((BR, D), lambda b: (b, 0)),
            pl.BlockSpec((1, 3), lambda b: (0, 0)),
        ],
        out_specs=pl.BlockSpec((BR, D), lambda b: (b, 0)),
        out_shape=jax.ShapeDtypeStruct((NE, D), jnp.float32),
    )(S2e, Ce, ee, w2)


def _k5u_body(s_ref, c_ref, emb_ref, out_ref):
    c = jnp.maximum(c_ref[...][:, 0:1], 1.0)
    out_ref[...] = s_ref[...] / c + emb_ref[...]


def _tc_user(S2u, Cu, ue):
    return pl.pallas_call(
        _k5u_body,
        grid=(NU // BR,),
        in_specs=[
            pl.BlockSpec((BR, D), lambda b: (b, 0)),
            pl.BlockSpec((BR, 16), lambda b: (b, 0)),
            pl.BlockSpec((BR, D), lambda b: (b, 0)),
        ],
        out_specs=pl.BlockSpec((BR, D), lambda b: (b, 0)),
        out_shape=jax.ShapeDtypeStruct((NU, D), jnp.float32),
    )(S2u, Cu, ue)


def kernel(entity_emb, user_emb, w, edge_index, edge_type, user_index,
           item_index):
    ee, ue = entity_emb, user_emb
    headS = edge_index[0].reshape(NS, NSC, SCH, CH)
    tailS = edge_index[1].reshape(NS, NSC, SCH, CH)
    tyS = edge_type.reshape(NS, NSC, SCH, CH)
    uidxS = user_index.reshape(NS, NSC, SCH, CH)
    iidxS = item_index.reshape(NS, NSC, SCH, CH)
    headW = edge_index[0].reshape(NS * NC, TCH_W, CH)
    tailW = edge_index[1].reshape(NS * NC, TCH_W, CH)
    tyW = edge_type.reshape(NS * NC, TCH_W, CH)
    uidxW = user_index.reshape(NS * NC, TCH_W, CH)
    iidxW = item_index.reshape(NS * NC, TCH_W, CH)
    embQ = ee.reshape(NE, 4, Q).transpose(1, 0, 2).reshape(4 * NE, Q)

    cnt = _sc_counts(headS, tyS, uidxS)
    sum_q = _sc_sums(embQ, headS, tailS, tyS, uidxS, iidxS)
    Sp = sum_q.transpose(1, 0, 2).reshape(NROW, D)
    S = jnp.concatenate([Sp[:3 * NE], Sp[UBASE:UBASE + NU]])
    C = jnp.concatenate([cnt[:3 * NE], cnt[UBASE:UBASE + NU]])
    EMB2 = jnp.stack([ee, ue])
    U_tbl = _tc_squash(S, C, EMB2)

    ee16 = ee.astype(jnp.bfloat16)
    simE, simU = _sc_sims(U_tbl, ee16, headW, tailW, tyW, uidxW, iidxW)

    sum2_q = _sc_wsums(embQ, headS, tailS, tyS, uidxS, iidxS,
                       simE.reshape(NS, NSC, SCH, CH),
                       simU.reshape(NS, NSC, SCH, CH))
    S2 = sum2_q.transpose(1, 0, 2).reshape(NROW, D)

    entity_agg = _tc_entity(S2[:3 * NE].reshape(3, NE, D),
                            C[:3 * NE].reshape(3, NE, 16),
                            ee, w.reshape(1, 3))
    user_agg = _tc_user(S2[UBASE:UBASE + NU], C[3 * NE:], ue)
    return (entity_agg, user_agg)
